# baseline (speedup 1.0000x reference)
import jax
import jax.numpy as jnp
from jax.experimental import pallas as pl
from jax.experimental.pallas import tpu as pltpu

B, CIN, COUT, H, W, K = 2, 96, 96, 384, 384, 3
TH = 32
NB = H // TH
N = TH * W


def _conv_body(w_ref, b_ref, x_ref, m_ref, o_ref, xs_ref):
    i = pl.program_id(1)
    flat = x_ref[0].reshape(CIN, N).astype(jnp.bfloat16)

    @pl.when(i != NB)
    def _():
        xs_ref[:, (TH + 1) * W:] = flat[:, :W]

    @pl.when(i == NB)
    def _():
        xs_ref[:, (TH + 1) * W:] = jnp.zeros((CIN, W), jnp.bfloat16)

    @pl.when(i != 0)
    def _():
        nf = (TH + 2) * W
        colf = jax.lax.broadcasted_iota(jnp.int32, (1, nf), 1) % W
        xs = xs_ref[...]
        xsl = pltpu.roll(jnp.where(colf == W - 1, jnp.bfloat16(0), xs),
                         1, axis=1)
        xsr = pltpu.roll(jnp.where(colf == 0, jnp.bfloat16(0), xs),
                         nf - 1, axis=1)
        acc = jnp.zeros((COUT, N), jnp.float32)
        for ky in range(K):
            for kx, src in ((0, xsl), (1, xs), (2, xsr)):
                acc = acc + jnp.dot(w_ref[ky * K + kx],
                                    src[:, ky * W: ky * W + N],
                                    preferred_element_type=jnp.float32)
        m = m_ref[0].reshape(1, N).astype(jnp.float32)
        res = (acc + b_ref[...]) * m
        o_ref[0] = res.reshape(COUT, TH, W)

    @pl.when(i == 0)
    def _():
        xs_ref[:, :W] = jnp.zeros((CIN, W), jnp.bfloat16)

    @pl.when((i != 0) & (i != NB))
    def _():
        xs_ref[:, :W] = xs_ref[:, TH * W: (TH + 1) * W]

    @pl.when(i != NB)
    def _():
        xs_ref[:, W: (TH + 1) * W] = flat


@jax.jit
def kernel(x, mask, weight, bias):
    mg = mask.reshape(B, H, W)
    wt = (weight.transpose(2, 3, 0, 1)
          .reshape(K * K, COUT, CIN).astype(jnp.bfloat16))
    b2 = bias.reshape(COUT, 1)

    return pl.pallas_call(
        _conv_body,
        grid=(B, NB + 1),
        in_specs=[
            pl.BlockSpec((K * K, COUT, CIN), lambda b, i: (0, 0, 0)),
            pl.BlockSpec((COUT, 1), lambda b, i: (0, 0)),
            pl.BlockSpec((1, CIN, TH, W),
                         lambda b, i: (b, 0, jnp.minimum(i, NB - 1), 0)),
            pl.BlockSpec((1, TH, W),
                         lambda b, i: (b, jnp.maximum(i - 1, 0), 0)),
        ],
        out_specs=pl.BlockSpec((1, COUT, TH, W),
                               lambda b, i: (b, 0, jnp.maximum(i - 1, 0), 0)),
        out_shape=jax.ShapeDtypeStruct((B, COUT, H, W), jnp.float32),
        scratch_shapes=[pltpu.VMEM((CIN, (TH + 2) * W), jnp.bfloat16)],
        compiler_params=pltpu.CompilerParams(
            dimension_semantics=("arbitrary", "arbitrary")),
    )(wt, b2, x, mg)
